# baseline (speedup 1.0000x reference)
import functools

import jax
import jax.numpy as jnp
from jax import lax
from jax.experimental import pallas as pl
from jax.experimental.pallas import tpu as pltpu
from jax.experimental.pallas import tpu_sc as plsc

_NUM_EMBEDDINGS = 1000000
_FEATURES = 64
_ROWS, _COLS = 16384, 50

_NC = 2
_NS = 16
_NW = _NC * _NS
_RPW = _ROWS // _NW
_NB = 4
_NGROUP = _RPW // _NB

_mesh = plsc.VectorSubcoreMesh(core_axis_name="c", subcore_axis_name="s")


@functools.partial(
    pl.kernel,
    mesh=_mesh,
    out_type=jax.ShapeDtypeStruct((_ROWS, _COLS, _FEATURES), jnp.float32),
    scratch_types=[
        [pltpu.VMEM((_COLS,), jnp.int32) for _ in range(_NB)],
        [pltpu.VMEM((_COLS, _FEATURES), jnp.float32) for _ in range(_NB)],
        [pltpu.SemaphoreType.DMA for _ in range(_NB)],
        [pltpu.SemaphoreType.DMA for _ in range(_NB)],
        [pltpu.SemaphoreType.DMA for _ in range(_NB)],
    ],
    compiler_params=pltpu.CompilerParams(use_tc_tiling_on_sc=False),
)
def _embed_sc(tok_hbm, table_hbm, out_hbm, idxs, bufs, isems, gsems, ssems):
    wid = lax.axis_index("s") * _NC + lax.axis_index("c")
    base = wid * _RPW

    def stage_idx(r, b):
        pltpu.async_copy(tok_hbm.at[base + r], idxs[b], isems[b])

    def wait_idx(b):
        pltpu.make_async_copy(tok_hbm.at[base], idxs[b], isems[b]).wait()

    def fire_gather(b):
        pltpu.async_copy(table_hbm.at[idxs[b]], bufs[b], gsems[b])

    def drain_gather(b):
        pltpu.make_async_copy(
            table_hbm.at[idxs[b]], bufs[b], gsems[b]).wait()

    def drain_write(b):
        pltpu.make_async_copy(bufs[b], out_hbm.at[base], ssems[b]).wait()

    for b in range(_NB):
        stage_idx(b, b)
    for b in range(_NB):
        wait_idx(b)
        fire_gather(b)

    def body(i, _):
        r0 = i * _NB
        for b in range(_NB):
            drain_gather(b)
            pltpu.async_copy(bufs[b], out_hbm.at[base + r0 + b], ssems[b])

            @pl.when(i < _NGROUP - 1)
            def _():
                stage_idx(r0 + _NB + b, b)

        @pl.when(i < _NGROUP - 1)
        def _():
            for b in range(_NB):
                drain_write(b)
                wait_idx(b)
                fire_gather(b)

        return 0

    lax.fori_loop(0, _NGROUP, body, 0)
    for b in range(_NB):
        drain_write(b)


def kernel(tokens, embedding):
    return _embed_sc(tokens.astype(jnp.int32), embedding)


# speedup vs baseline: 1.0411x; 1.0411x over previous
import functools

import jax
import jax.numpy as jnp
from jax import lax
from jax.experimental import pallas as pl
from jax.experimental.pallas import tpu as pltpu
from jax.experimental.pallas import tpu_sc as plsc

_NUM_EMBEDDINGS = 1000000
_FEATURES = 64
_ROWS, _COLS = 16384, 50

_NC = 2
_NS = 16
_NW = _NC * _NS
_RPW = _ROWS // _NW
_NB = 8
_NGROUP = _RPW // _NB

_mesh = plsc.VectorSubcoreMesh(core_axis_name="c", subcore_axis_name="s")


@functools.partial(
    pl.kernel,
    mesh=_mesh,
    out_type=jax.ShapeDtypeStruct((_ROWS, _COLS, _FEATURES), jnp.float32),
    scratch_types=[
        [pltpu.VMEM((_COLS,), jnp.int32) for _ in range(_NB)],
        [pltpu.VMEM((_COLS, _FEATURES), jnp.float32) for _ in range(_NB)],
        [pltpu.SemaphoreType.DMA for _ in range(_NB)],
        [pltpu.SemaphoreType.DMA for _ in range(_NB)],
        [pltpu.SemaphoreType.DMA for _ in range(_NB)],
    ],
    compiler_params=pltpu.CompilerParams(use_tc_tiling_on_sc=False),
)
def _embed_sc(tok_hbm, table_hbm, out_hbm, idxs, bufs, isems, gsems, ssems):
    wid = lax.axis_index("s") * _NC + lax.axis_index("c")
    base = wid * _RPW

    def stage_idx(r, b):
        pltpu.async_copy(tok_hbm.at[base + r], idxs[b], isems[b])

    def wait_idx(b):
        pltpu.make_async_copy(tok_hbm.at[base], idxs[b], isems[b]).wait()

    def fire_gather(b):
        pltpu.async_copy(table_hbm.at[idxs[b]], bufs[b], gsems[b])

    def drain_gather(b):
        pltpu.make_async_copy(
            table_hbm.at[idxs[b]], bufs[b], gsems[b]).wait()

    def drain_write(b):
        pltpu.make_async_copy(bufs[b], out_hbm.at[base], ssems[b]).wait()

    for b in range(_NB):
        stage_idx(b, b)
    for b in range(_NB):
        wait_idx(b)
        fire_gather(b)

    def body(i, _):
        r0 = i * _NB
        for b in range(_NB):
            drain_gather(b)
            pltpu.async_copy(bufs[b], out_hbm.at[base + r0 + b], ssems[b])

            @pl.when(i < _NGROUP - 1)
            def _():
                stage_idx(r0 + _NB + b, b)

        @pl.when(i < _NGROUP - 1)
        def _():
            for b in range(_NB):
                drain_write(b)
                wait_idx(b)
                fire_gather(b)

        return 0

    lax.fori_loop(0, _NGROUP, body, 0)
    for b in range(_NB):
        drain_write(b)


def kernel(tokens, embedding):
    return _embed_sc(tokens.astype(jnp.int32), embedding)


# speedup vs baseline: 1.0456x; 1.0043x over previous
import functools

import jax
import jax.numpy as jnp
from jax import lax
from jax.experimental import pallas as pl
from jax.experimental.pallas import tpu as pltpu
from jax.experimental.pallas import tpu_sc as plsc

_NUM_EMBEDDINGS = 1000000
_FEATURES = 64
_ROWS, _COLS = 16384, 50
_FLAT = _ROWS * _COLS

_NC = 2
_NS = 16
_NW = _NC * _NS
_CHUNK = 128
_NCHUNK = _FLAT // _CHUNK // _NW
_NB = 8
_NG = _NCHUNK // _NB

_mesh = plsc.VectorSubcoreMesh(core_axis_name="c", subcore_axis_name="s")


@functools.partial(
    pl.kernel,
    mesh=_mesh,
    out_type=jax.ShapeDtypeStruct((_FLAT, _FEATURES), jnp.float32),
    scratch_types=[
        pltpu.VMEM((_NCHUNK, _CHUNK), jnp.int32),
        [pltpu.VMEM((_CHUNK, _FEATURES), jnp.float32) for _ in range(_NB)],
        pltpu.SemaphoreType.DMA,
        [pltpu.SemaphoreType.DMA for _ in range(_NB)],
        [pltpu.SemaphoreType.DMA for _ in range(_NB)],
    ],
    compiler_params=pltpu.CompilerParams(use_tc_tiling_on_sc=False),
)
def _embed_sc(tok_hbm, table_hbm, out_hbm, idx, bufs, isem, gsems, ssems):
    wid = lax.axis_index("s") * _NC + lax.axis_index("c")
    ibase = wid * _NCHUNK
    obase = wid * _NCHUNK * _CHUNK

    pltpu.async_copy(tok_hbm.at[pl.ds(ibase, _NCHUNK)], idx, isem)
    pltpu.make_async_copy(tok_hbm.at[pl.ds(ibase, _NCHUNK)], idx, isem).wait()

    def fire_gather(j, b):
        pltpu.async_copy(table_hbm.at[idx.at[j]], bufs[b], gsems[b])

    def drain_gather(b):
        pltpu.make_async_copy(
            table_hbm.at[idx.at[0]], bufs[b], gsems[b]).wait()

    def fire_write(j, b):
        pltpu.async_copy(
            bufs[b], out_hbm.at[pl.ds(obase + j * _CHUNK, _CHUNK)], ssems[b])

    def drain_write(b):
        pltpu.make_async_copy(
            bufs[b], out_hbm.at[pl.ds(obase, _CHUNK)], ssems[b]).wait()

    for b in range(_NB):
        fire_gather(b, b)

    def body(i, _):
        j0 = i * _NB
        for b in range(_NB):
            drain_gather(b)
            fire_write(j0 + b, b)

        @pl.when(i < _NG - 1)
        def _():
            for b in range(_NB):
                drain_write(b)
                fire_gather(j0 + _NB + b, b)

        return 0

    lax.fori_loop(0, _NG, body, 0)
    for b in range(_NB):
        drain_write(b)


def kernel(tokens, embedding):
    tok_flat = tokens.astype(jnp.int32).reshape(_FLAT // _CHUNK, _CHUNK)
    out = _embed_sc(tok_flat, embedding)
    return out.reshape(_ROWS, _COLS, _FEATURES)


# speedup vs baseline: 1.0458x; 1.0002x over previous
import functools

import jax
import jax.numpy as jnp
from jax import lax
from jax.experimental import pallas as pl
from jax.experimental.pallas import tpu as pltpu
from jax.experimental.pallas import tpu_sc as plsc

_NUM_EMBEDDINGS = 1000000
_FEATURES = 64
_ROWS, _COLS = 16384, 50
_FLAT = _ROWS * _COLS

_NC = 2
_NS = 16
_NW = _NC * _NS
_CHUNK = 128
_NCHUNK = _FLAT // _CHUNK // _NW
_NB = 10
_NG = _NCHUNK // _NB

_mesh = plsc.VectorSubcoreMesh(core_axis_name="c", subcore_axis_name="s")


@functools.partial(
    pl.kernel,
    mesh=_mesh,
    out_type=jax.ShapeDtypeStruct((_FLAT, _FEATURES), jnp.float32),
    scratch_types=[
        pltpu.VMEM((_NCHUNK, _CHUNK), jnp.int32),
        [pltpu.VMEM((_CHUNK, _FEATURES), jnp.float32) for _ in range(_NB)],
        pltpu.SemaphoreType.DMA,
        [pltpu.SemaphoreType.DMA for _ in range(_NB)],
        [pltpu.SemaphoreType.DMA for _ in range(_NB)],
    ],
    compiler_params=pltpu.CompilerParams(use_tc_tiling_on_sc=False),
)
def _embed_sc(tok_hbm, table_hbm, out_hbm, idx, bufs, isem, gsems, ssems):
    wid = lax.axis_index("s") * _NC + lax.axis_index("c")
    ibase = wid * _NCHUNK
    obase = wid * _NCHUNK * _CHUNK

    pltpu.async_copy(tok_hbm.at[pl.ds(ibase, _NCHUNK)], idx, isem)
    pltpu.make_async_copy(tok_hbm.at[pl.ds(ibase, _NCHUNK)], idx, isem).wait()

    def fire_gather(j, b):
        pltpu.async_copy(table_hbm.at[idx.at[j]], bufs[b], gsems[b])

    def drain_gather(b):
        pltpu.make_async_copy(
            table_hbm.at[idx.at[0]], bufs[b], gsems[b]).wait()

    def fire_write(j, b):
        pltpu.async_copy(
            bufs[b], out_hbm.at[pl.ds(obase + j * _CHUNK, _CHUNK)], ssems[b])

    def drain_write(b):
        pltpu.make_async_copy(
            bufs[b], out_hbm.at[pl.ds(obase, _CHUNK)], ssems[b]).wait()

    for b in range(_NB):
        fire_gather(b, b)

    def body(i, _):
        j0 = i * _NB
        for b in range(_NB):
            drain_gather(b)
            fire_write(j0 + b, b)

        @pl.when(i < _NG - 1)
        def _():
            for b in range(_NB):
                drain_write(b)
                fire_gather(j0 + _NB + b, b)

        return 0

    lax.fori_loop(0, _NG, body, 0)
    for b in range(_NB):
        drain_write(b)


def kernel(tokens, embedding):
    tok_flat = tokens.astype(jnp.int32).reshape(_FLAT // _CHUNK, _CHUNK)
    out = _embed_sc(tok_flat, embedding)
    return out.reshape(_ROWS, _COLS, _FEATURES)


# speedup vs baseline: 1.0502x; 1.0041x over previous
import functools

import jax
import jax.numpy as jnp
from jax import lax
from jax.experimental import pallas as pl
from jax.experimental.pallas import tpu as pltpu
from jax.experimental.pallas import tpu_sc as plsc

_NUM_EMBEDDINGS = 1000000
_FEATURES = 64
_ROWS, _COLS = 16384, 50
_FLAT = _ROWS * _COLS

_NC = 2
_NS = 16
_NW = _NC * _NS
_CHUNK = 128
_NCHUNK = _FLAT // _CHUNK // _NW
_NB = 8
_NSB = 4
_NG = _NCHUNK // _NB

_mesh = plsc.VectorSubcoreMesh(core_axis_name="c", subcore_axis_name="s")


@functools.partial(
    pl.kernel,
    mesh=_mesh,
    out_type=jax.ShapeDtypeStruct((_FLAT, _FEATURES), jnp.float32),
    scratch_types=[
        pltpu.VMEM((_NCHUNK, _CHUNK), jnp.int32),
        [pltpu.VMEM((_CHUNK, _FEATURES), jnp.float32) for _ in range(_NB)],
        pltpu.VMEM_SHARED((_NS, _NSB, _CHUNK, _FEATURES), jnp.float32),
        pltpu.SemaphoreType.DMA,
        [pltpu.SemaphoreType.DMA for _ in range(_NB)],
        [pltpu.SemaphoreType.DMA for _ in range(_NSB)],
        [pltpu.SemaphoreType.DMA for _ in range(_NSB)],
    ],
    compiler_params=pltpu.CompilerParams(use_tc_tiling_on_sc=False),
)
def _embed_sc(tok_hbm, table_hbm, out_hbm, idx, bufs, spmem, isem, gsems,
              lsems, wsems):
    sid = lax.axis_index("s")
    wid = sid * _NC + lax.axis_index("c")
    ibase = wid * _NCHUNK
    obase = wid * _NCHUNK * _CHUNK

    pltpu.async_copy(tok_hbm.at[pl.ds(ibase, _NCHUNK)], idx, isem)
    pltpu.make_async_copy(tok_hbm.at[pl.ds(ibase, _NCHUNK)], idx, isem).wait()

    def fire_gather(j, b):
        pltpu.async_copy(table_hbm.at[idx.at[j]], bufs[b], gsems[b])

    def drain_gather(b):
        pltpu.make_async_copy(
            table_hbm.at[idx.at[0]], bufs[b], gsems[b]).wait()

    def fire_local(b, sb):
        pltpu.async_copy(bufs[b], spmem.at[sid, sb], lsems[sb])

    def drain_local(b, sb):
        pltpu.make_async_copy(bufs[b], spmem.at[sid, sb], lsems[sb]).wait()

    def fire_write(j, sb):
        pltpu.async_copy(
            spmem.at[sid, sb],
            out_hbm.at[pl.ds(obase + j * _CHUNK, _CHUNK)], wsems[sb])

    def drain_write(sb):
        pltpu.make_async_copy(
            spmem.at[sid, sb], out_hbm.at[pl.ds(obase, _CHUNK)],
            wsems[sb]).wait()

    for b in range(_NB):
        fire_gather(b, b)

    def body(i, _):
        j0 = i * _NB
        for b in range(_NB):
            sb = b % _NSB
            drain_gather(b)

            if b >= _NSB:
                drain_write(sb)
            else:

                @pl.when(i > 0)
                def _():
                    drain_write(sb)

            fire_local(b, sb)
            drain_local(b, sb)
            fire_write(j0 + b, sb)

            @pl.when(i < _NG - 1)
            def _():
                fire_gather(j0 + _NB + b, b)

        return 0

    lax.fori_loop(0, _NG, body, 0)
    for sb in range(_NSB):
        drain_write(sb)


def kernel(tokens, embedding):
    tok_flat = tokens.astype(jnp.int32).reshape(_FLAT // _CHUNK, _CHUNK)
    out = _embed_sc(tok_flat, embedding)
    return out.reshape(_ROWS, _COLS, _FEATURES)
